# Initial kernel scaffold; baseline (speedup 1.0000x reference)
import functools

import jax
import jax.numpy as jnp
from jax import lax
from jax.experimental import pallas as pl
from jax.experimental.pallas import tpu as pltpu
from jax.experimental.pallas import tpu_sc as plsc

N = 10000
E = 160000
DIN = 256
DH = 128
NS = 16
L = 16
EPT = E // NS
CH = 80
NCHUNK = EPT // CH
ROWS_PT = N // NS


def _matmul_stacked(x, W):
    def body(x_ref, w_ref, o_ref):
        o_ref[...] = lax.dot_general(
            x_ref[...], w_ref[...],
            dimension_numbers=(((1,), (1,)), ((), ())),
            preferred_element_type=jnp.float32)

    return pl.pallas_call(
        body,
        grid=(2,),
        in_specs=[
            pl.BlockSpec((N, DIN), lambda c: (0, 0)),
            pl.BlockSpec((DH, DIN), lambda c: (c, 0)),
        ],
        out_specs=pl.BlockSpec((N, DH), lambda c: (c, 0)),
        out_shape=jax.ShapeDtypeStruct((2 * N, DH), jnp.float32),
    )(x, W)


def _make_spmm():
    mesh = plsc.VectorSubcoreMesh(core_axis_name="c", subcore_axis_name="s")

    @functools.partial(
        pl.kernel,
        out_type=jax.ShapeDtypeStruct((2 * N, DH), jnp.float32),
        mesh=mesh,
        scratch_types=[
            pltpu.VMEM((NCHUNK, CH), jnp.int32),
            pltpu.VMEM((NCHUNK, CH), jnp.int32),
            pltpu.VMEM((NCHUNK, CH), jnp.float32),
            pltpu.VMEM((CH, DH), jnp.float32),
            pltpu.VMEM_SHARED((N, DH), jnp.float32),
            pltpu.SemaphoreType.DMA,
        ],
    )
    def spmm(h_hbm, src_hbm, dst_hbm, w_hbm, out_hbm,
             src_v, dst_v, w_v, msgs_v, acc_sh, sem):
        c = lax.axis_index("c")
        s = lax.axis_index("s")

        pltpu.sync_copy(src_hbm.at[s], src_v)
        pltpu.sync_copy(dst_hbm.at[s], dst_v)
        pltpu.sync_copy(w_hbm.at[s], w_v)

        coff = jnp.full((L,), c * N, dtype=jnp.int32)

        def off_row(k, carry):
            for t in range(CH // L):
                sl = (k, pl.ds(t * L, L))
                src_v[sl] = src_v[sl] + coff
            return carry
        lax.fori_loop(0, NCHUNK, off_row, 0)

        zeros = jnp.zeros((L,), jnp.float32)

        def zero_row(i, carry):
            for j in range(DH // L):
                msgs_v[i, pl.ds(j * L, L)] = zeros
            return carry
        lax.fori_loop(0, CH, zero_row, 0)

        nfull = ROWS_PT // CH
        rem = ROWS_PT - nfull * CH

        def zero_dma(t, carry):
            pltpu.sync_copy(msgs_v, acc_sh.at[pl.ds(s * ROWS_PT + t * CH, CH)])
            return carry
        lax.fori_loop(0, nfull, zero_dma, 0)
        if rem:
            pltpu.sync_copy(msgs_v.at[pl.ds(0, rem)],
                            acc_sh.at[pl.ds(s * ROWS_PT + nfull * CH, rem)])
        plsc.subcore_barrier()

        def chunk(k, carry):
            pltpu.async_copy(h_hbm.at[src_v.at[k]], msgs_v, sem).wait()

            def mul_row(i, cc):
                wb = plsc.load_gather(
                    w_v, [jnp.full((L,), k, jnp.int32),
                          jnp.full((L,), i, jnp.int32)])
                for j in range(DH // L):
                    sl = (i, pl.ds(j * L, L))
                    msgs_v[sl] = msgs_v[sl] * wb
                return cc
            lax.fori_loop(0, CH, mul_row, 0)

            pltpu.sync_copy(msgs_v, acc_sh.at[dst_v.at[k]], add=True)
            return carry
        lax.fori_loop(0, NCHUNK, chunk, 0)

        plsc.subcore_barrier()

        pltpu.sync_copy(acc_sh.at[pl.ds(s * ROWS_PT, ROWS_PT)],
                        out_hbm.at[pl.ds(c * N + s * ROWS_PT, ROWS_PT)])

    return spmm


_spmm = _make_spmm()


def kernel(x, edge_index, edge_weight, W):
    dst = edge_index[0].astype(jnp.int32).reshape(NS, NCHUNK, CH)
    src = edge_index[1].astype(jnp.int32).reshape(NS, NCHUNK, CH)
    w3 = edge_weight.reshape(NS, NCHUNK, CH)
    h = _matmul_stacked(x, W)
    o = _spmm(h, src, dst, w3)
    return jnp.concatenate([o[:N], o[N:]], axis=1)


# speedup vs baseline: 3.0791x; 3.0791x over previous
import functools

import jax
import jax.numpy as jnp
from jax import lax
from jax.experimental import pallas as pl
from jax.experimental.pallas import tpu as pltpu
from jax.experimental.pallas import tpu_sc as plsc

N = 10000
E = 160000
DIN = 256
DH = 128
NS = 16
L = 16
EPT = E // NS
CH = 80
NCHUNK = EPT // CH
ROWS_PT = 624
ROWS_REM = N - ROWS_PT * NS


def _matmul_stacked(x, W):
    def body(x_ref, w_ref, o_ref):
        o_ref[...] = lax.dot_general(
            x_ref[...], w_ref[...],
            dimension_numbers=(((1,), (1,)), ((), ())),
            preferred_element_type=jnp.float32)

    return pl.pallas_call(
        body,
        grid=(2,),
        in_specs=[
            pl.BlockSpec((N, DIN), lambda c: (0, 0)),
            pl.BlockSpec((DH, DIN), lambda c: (c, 0)),
        ],
        out_specs=pl.BlockSpec((N, DH), lambda c: (c, 0)),
        out_shape=jax.ShapeDtypeStruct((2 * N, DH), jnp.float32),
    )(x, W)


def _make_spmm():
    mesh = plsc.VectorSubcoreMesh(core_axis_name="c", subcore_axis_name="s")

    @functools.partial(
        pl.kernel,
        out_type=jax.ShapeDtypeStruct((2 * N, DH), jnp.float32),
        mesh=mesh,
        scratch_types=[
            pltpu.VMEM((2, CH), jnp.int32),
            pltpu.VMEM((CH,), jnp.float32),
            pltpu.VMEM((CH, DH), jnp.float32),
            pltpu.VMEM_SHARED((N, DH), jnp.float32),
            pltpu.SemaphoreType.DMA,
        ],
    )
    def spmm(h_hbm, idx_hbm, w_hbm, out_hbm,
             idx_v, w_v, msgs_v, acc_sh, sem):
        c = lax.axis_index("c")
        s = lax.axis_index("s")

        zeros = jnp.zeros((L,), jnp.float32)

        def zero_row(i, carry):
            for j in range(DH // L):
                msgs_v[i, pl.ds(j * L, L)] = zeros
            return carry
        lax.fori_loop(0, CH, zero_row, 0)

        nfull = ROWS_PT // CH
        rem = ROWS_PT - nfull * CH

        def zero_dma(t, carry):
            pltpu.sync_copy(msgs_v, acc_sh.at[pl.ds(s * ROWS_PT + t * CH, CH)])
            return carry
        lax.fori_loop(0, nfull, zero_dma, 0)
        if rem:
            pltpu.sync_copy(msgs_v.at[pl.ds(0, rem)],
                            acc_sh.at[pl.ds(s * ROWS_PT + nfull * CH, rem)])

        @pl.when(s == 0)
        def _():
            pltpu.sync_copy(msgs_v.at[pl.ds(0, ROWS_REM)],
                            acc_sh.at[pl.ds(ROWS_PT * NS, ROWS_REM)])
        plsc.subcore_barrier()

        def chunk(k, carry):
            pltpu.sync_copy(idx_hbm.at[c, s, k], idx_v)
            pltpu.sync_copy(w_hbm.at[s, k], w_v)
            pltpu.async_copy(h_hbm.at[idx_v.at[0]], msgs_v, sem).wait()

            def mul_grp(g, cc):
                wv16 = w_v[pl.ds(g * L, L)]
                for i2 in range(L):
                    w = wv16[i2]
                    row = g * L + i2
                    for j in range(DH // L):
                        sl = (row, pl.ds(j * L, L))
                        msgs_v[sl] = msgs_v[sl] * w
                return cc
            lax.fori_loop(0, CH // L, mul_grp, 0)

            pltpu.sync_copy(msgs_v, acc_sh.at[idx_v.at[1]], add=True)
            return carry
        lax.fori_loop(0, NCHUNK, chunk, 0)

        plsc.subcore_barrier()

        pltpu.sync_copy(acc_sh.at[pl.ds(s * ROWS_PT, ROWS_PT)],
                        out_hbm.at[pl.ds(c * N + s * ROWS_PT, ROWS_PT)])

        @pl.when(s == 0)
        def _():
            pltpu.sync_copy(acc_sh.at[pl.ds(ROWS_PT * NS, ROWS_REM)],
                            out_hbm.at[pl.ds(c * N + ROWS_PT * NS, ROWS_REM)])

    return spmm


_spmm = _make_spmm()


def kernel(x, edge_index, edge_weight, W):
    dst = edge_index[0].astype(jnp.int32)
    src = edge_index[1].astype(jnp.int32)
    sd0 = jnp.stack([src, dst]).reshape(2, NS, NCHUNK, CH).transpose(1, 2, 0, 3)
    sd1 = jnp.stack([src + N, dst]).reshape(2, NS, NCHUNK, CH).transpose(1, 2, 0, 3)
    idx_all = jnp.stack([sd0, sd1])
    w3 = edge_weight.reshape(NS, NCHUNK, CH)
    h = _matmul_stacked(x, W)
    o = _spmm(h, idx_all, w3)
    return jnp.concatenate([o[:N], o[N:]], axis=1)


# speedup vs baseline: 4.2695x; 1.3866x over previous
import functools

import jax
import jax.numpy as jnp
from jax import lax
from jax.experimental import pallas as pl
from jax.experimental.pallas import tpu as pltpu
from jax.experimental.pallas import tpu_sc as plsc

N = 10000
E = 160000
DIN = 256
DH = 128
NS = 16
L = 16
CH = 64
NSUP = 79
EPT = NSUP * 2 * CH
E_PAD = NS * EPT
ROWS_PT = 624
ROWS_REM = N - ROWS_PT * NS


def _matmul_stacked(x, W):
    def body(x_ref, w_ref, o_ref):
        o_ref[...] = lax.dot_general(
            x_ref[...], w_ref[...],
            dimension_numbers=(((1,), (1,)), ((), ())),
            preferred_element_type=jnp.float32)

    return pl.pallas_call(
        body,
        grid=(2,),
        in_specs=[
            pl.BlockSpec((N, DIN), lambda c: (0, 0)),
            pl.BlockSpec((DH, DIN), lambda c: (c, 0)),
        ],
        out_specs=pl.BlockSpec((N, DH), lambda c: (c, 0)),
        out_shape=jax.ShapeDtypeStruct((2 * N, DH), jnp.float32),
    )(x, W)


def _make_spmm():
    mesh = plsc.VectorSubcoreMesh(core_axis_name="c", subcore_axis_name="s")

    @functools.partial(
        pl.kernel,
        out_type=jax.ShapeDtypeStruct((2 * N, DH), jnp.float32),
        mesh=mesh,
        scratch_types=[
            pltpu.VMEM((4, CH), jnp.int32),
            pltpu.VMEM((4, CH), jnp.int32),
            pltpu.VMEM((2, CH), jnp.float32),
            pltpu.VMEM((2, CH), jnp.float32),
            pltpu.VMEM((2 * CH, DH), jnp.float32),
            pltpu.VMEM((2 * CH, DH), jnp.float32),
            pltpu.VMEM_SHARED((N, DH), jnp.float32),
            pltpu.SemaphoreType.DMA,
            pltpu.SemaphoreType.DMA,
            pltpu.SemaphoreType.DMA,
            pltpu.SemaphoreType.DMA,
        ],
    )
    def spmm(h_hbm, idx_hbm, w_hbm, out_hbm,
             idx0, idx1, w0, w1, m0, m1, acc_sh, sg0, sg1, ss0, ss1):
        c = lax.axis_index("c")
        s = lax.axis_index("s")
        idx_r = (idx0, idx1)
        w_r = (w0, w1)
        msgs = (m0, m1)
        sem_g = (sg0, sg1)
        sem_s = (ss0, ss1)

        def stage(S, b):
            pltpu.sync_copy(idx_hbm.at[c, s, S], idx_r[b])
            pltpu.sync_copy(w_hbm.at[s, S], w_r[b])

        def issue_gathers(b):
            for h in (0, 1):
                pltpu.async_copy(h_hbm.at[idx_r[b].at[2 * h]],
                                 msgs[b].at[pl.ds(h * CH, CH)], sem_g[b])

        def wait_gathers(b):
            for h in (0, 1):
                pltpu.make_async_copy(h_hbm.at[idx_r[b].at[2 * h]],
                                      msgs[b].at[pl.ds(h * CH, CH)],
                                      sem_g[b]).wait()

        def issue_scatter(b, h):
            pltpu.async_copy(msgs[b].at[pl.ds(h * CH, CH)],
                             acc_sh.at[idx_r[b].at[2 * h + 1]],
                             sem_s[b], add=True)

        def wait_scatters(b):
            for h in (0, 1):
                pltpu.make_async_copy(msgs[b].at[pl.ds(h * CH, CH)],
                                      acc_sh.at[idx_r[b].at[2 * h + 1]],
                                      sem_s[b]).wait()

        def mul_half(b, h):
            def grp(g, cc):
                wv16 = w_r[b][h, pl.ds(g * L, L)]
                for i2 in range(L):
                    w = wv16[i2]
                    row = h * CH + g * L + i2
                    for j in range(DH // L):
                        sl = (row, pl.ds(j * L, L))
                        msgs[b][sl] = msgs[b][sl] * w
                return cc
            lax.fori_loop(0, CH // L, grp, 0)

        def section(S, b, prep, wait_prev):
            other = 1 - b
            if prep:
                if wait_prev:
                    wait_scatters(other)
                stage(S + 1, other)
                issue_gathers(other)
            wait_gathers(b)
            for h in (0, 1):
                mul_half(b, h)
                issue_scatter(b, h)

        zeros = jnp.zeros((L,), jnp.float32)

        def zero_row(i, carry):
            for j in range(DH // L):
                m0[i, pl.ds(j * L, L)] = zeros
            return carry
        lax.fori_loop(0, 2 * CH, zero_row, 0)

        ZR = 2 * CH
        nfull = ROWS_PT // ZR
        rem = ROWS_PT - nfull * ZR

        def zero_dma(t, carry):
            pltpu.sync_copy(m0, acc_sh.at[pl.ds(s * ROWS_PT + t * ZR, ZR)])
            return carry
        lax.fori_loop(0, nfull, zero_dma, 0)
        if rem:
            pltpu.sync_copy(m0.at[pl.ds(0, rem)],
                            acc_sh.at[pl.ds(s * ROWS_PT + nfull * ZR, rem)])

        @pl.when(s == 0)
        def _():
            pltpu.sync_copy(m0.at[pl.ds(0, ROWS_REM)],
                            acc_sh.at[pl.ds(ROWS_PT * NS, ROWS_REM)])
        plsc.subcore_barrier()

        stage(0, 0)
        issue_gathers(0)
        section(0, 0, prep=True, wait_prev=False)

        def pair(g, carry):
            section(2 * g + 1, 1, prep=True, wait_prev=True)
            section(2 * g + 2, 0, prep=True, wait_prev=True)
            return carry
        lax.fori_loop(0, (NSUP - 3) // 2, pair, 0)

        section(NSUP - 2, 1, prep=True, wait_prev=True)
        section(NSUP - 1, 0, prep=False, wait_prev=False)
        wait_scatters(1)
        wait_scatters(0)

        plsc.subcore_barrier()

        pltpu.sync_copy(acc_sh.at[pl.ds(s * ROWS_PT, ROWS_PT)],
                        out_hbm.at[pl.ds(c * N + s * ROWS_PT, ROWS_PT)])

        @pl.when(s == 0)
        def _():
            pltpu.sync_copy(acc_sh.at[pl.ds(ROWS_PT * NS, ROWS_REM)],
                            out_hbm.at[pl.ds(c * N + ROWS_PT * NS, ROWS_REM)])

    return spmm


_spmm = _make_spmm()


def kernel(x, edge_index, edge_weight, W):
    dst = edge_index[0].astype(jnp.int32)
    src = edge_index[1].astype(jnp.int32)
    pad = E_PAD - E
    dst_p = jnp.concatenate([dst, jnp.zeros((pad,), jnp.int32)])
    src_p = jnp.concatenate([src, jnp.zeros((pad,), jnp.int32)])
    w_p = jnp.concatenate([edge_weight, jnp.zeros((pad,), jnp.float32)])

    dstr = dst_p.reshape(NS, NSUP, 2, CH)
    w_hbm = w_p.reshape(NS, NSUP, 2, CH)

    def pack(srcr):
        return jnp.stack([srcr[:, :, 0], dstr[:, :, 0],
                          srcr[:, :, 1], dstr[:, :, 1]], axis=2)

    s0 = src_p.reshape(NS, NSUP, 2, CH)
    idx_all = jnp.stack([pack(s0), pack(s0 + N)])

    h = _matmul_stacked(x, W)
    o = _spmm(h, idx_all, w_hbm)
    return jnp.concatenate([o[:N], o[N:]], axis=1)


# speedup vs baseline: 4.9884x; 1.1684x over previous
import functools

import jax
import jax.numpy as jnp
from jax import lax
from jax.experimental import pallas as pl
from jax.experimental.pallas import tpu as pltpu
from jax.experimental.pallas import tpu_sc as plsc

N = 10000
E = 160000
DIN = 256
DH = 128
NS = 16
L = 16
CH = 64
NSUP = 79
EPT = NSUP * 2 * CH
E_PAD = NS * EPT
ROWS_PT = 624
ROWS_REM = N - ROWS_PT * NS
NIDX = 4
NMSG = 2


def _matmul_stacked(x, W):
    def body(x_ref, w_ref, o_ref):
        o_ref[...] = lax.dot_general(
            x_ref[...], w_ref[...],
            dimension_numbers=(((1,), (1,)), ((), ())),
            preferred_element_type=jnp.float32)

    return pl.pallas_call(
        body,
        grid=(2,),
        in_specs=[
            pl.BlockSpec((N, DIN), lambda c: (0, 0)),
            pl.BlockSpec((DH, DIN), lambda c: (c, 0)),
        ],
        out_specs=pl.BlockSpec((N, DH), lambda c: (c, 0)),
        out_shape=jax.ShapeDtypeStruct((2 * N, DH), jnp.float32),
    )(x, W)


def _make_spmm():
    mesh = plsc.VectorSubcoreMesh(core_axis_name="c", subcore_axis_name="s")

    @functools.partial(
        pl.kernel,
        out_type=jax.ShapeDtypeStruct((2 * N, DH), jnp.float32),
        mesh=mesh,
        scratch_types=(
            [pltpu.VMEM((4, CH), jnp.int32) for _ in range(NIDX)] +
            [pltpu.VMEM((2, CH), jnp.float32) for _ in range(NIDX)] +
            [pltpu.VMEM((2 * CH, DH), jnp.float32) for _ in range(NMSG)] +
            [pltpu.VMEM_SHARED((N, DH), jnp.float32)] +
            [pltpu.SemaphoreType.DMA for _ in range(NIDX + 2 * NMSG)]
        ),
    )
    def spmm(h_hbm, idx_hbm, w_hbm, out_hbm, *scr):
        idx_r = scr[0:NIDX]
        w_r = scr[NIDX:2 * NIDX]
        msgs = scr[2 * NIDX:2 * NIDX + NMSG]
        acc_sh = scr[2 * NIDX + NMSG]
        sems = scr[2 * NIDX + NMSG + 1:]
        sem_i = sems[0:NIDX]
        sem_g = sems[NIDX:NIDX + NMSG]
        sem_s = sems[NIDX + NMSG:]

        c = lax.axis_index("c")
        s = lax.axis_index("s")

        def stage_async(S, q):
            pltpu.async_copy(idx_hbm.at[c, s, S], idx_r[q], sem_i[q])
            pltpu.async_copy(w_hbm.at[s, S], w_r[q], sem_i[q])

        def wait_stage(q):
            pltpu.make_async_copy(idx_hbm.at[c, s, 0], idx_r[q], sem_i[q]).wait()
            pltpu.make_async_copy(w_hbm.at[s, 0], w_r[q], sem_i[q]).wait()

        def issue_gathers(b, q):
            for h in (0, 1):
                pltpu.async_copy(h_hbm.at[idx_r[q].at[2 * h]],
                                 msgs[b].at[pl.ds(h * CH, CH)], sem_g[b])

        def wait_gathers(b, q):
            for h in (0, 1):
                pltpu.make_async_copy(h_hbm.at[idx_r[q].at[2 * h]],
                                      msgs[b].at[pl.ds(h * CH, CH)],
                                      sem_g[b]).wait()

        def issue_scatter(b, q, h):
            pltpu.async_copy(msgs[b].at[pl.ds(h * CH, CH)],
                             acc_sh.at[idx_r[q].at[2 * h + 1]],
                             sem_s[b], add=True)

        def wait_scatters(b, q):
            for h in (0, 1):
                pltpu.make_async_copy(msgs[b].at[pl.ds(h * CH, CH)],
                                      acc_sh.at[idx_r[q].at[2 * h + 1]],
                                      sem_s[b]).wait()

        def mul_half(b, q, h):
            def grp(g, cc):
                wv16 = w_r[q][h, pl.ds(g * L, L)]
                for i2 in range(L):
                    w = wv16[i2]
                    row = h * CH + g * L + i2
                    for j in range(DH // L):
                        sl = (row, pl.ds(j * L, L))
                        msgs[b][sl] = msgs[b][sl] * w
                return cc
            lax.fori_loop(0, CH // L, grp, 0)

        def section(S, phase, prefetch, prep, wait_prev):
            b, q = phase % NMSG, phase % NIDX
            b1, q1 = (phase + 1) % NMSG, (phase + 1) % NIDX
            q2 = (phase + 2) % NIDX
            if prefetch:
                stage_async(S + 2, q2)
            if prep:
                if wait_prev:
                    wait_scatters(b1, q1)
                wait_stage(q1)
                issue_gathers(b1, q1)
            wait_gathers(b, q)
            for h in (0, 1):
                mul_half(b, q, h)
                issue_scatter(b, q, h)

        zeros = jnp.zeros((L,), jnp.float32)

        def zero_row(i, carry):
            for j in range(DH // L):
                msgs[0][i, pl.ds(j * L, L)] = zeros
            return carry
        lax.fori_loop(0, 2 * CH, zero_row, 0)

        ZR = 2 * CH
        nfull = ROWS_PT // ZR
        rem = ROWS_PT - nfull * ZR

        def zero_dma(t, carry):
            pltpu.sync_copy(msgs[0], acc_sh.at[pl.ds(s * ROWS_PT + t * ZR, ZR)])
            return carry
        lax.fori_loop(0, nfull, zero_dma, 0)
        if rem:
            pltpu.sync_copy(msgs[0].at[pl.ds(0, rem)],
                            acc_sh.at[pl.ds(s * ROWS_PT + nfull * ZR, rem)])

        @pl.when(s == 0)
        def _():
            pltpu.sync_copy(msgs[0].at[pl.ds(0, ROWS_REM)],
                            acc_sh.at[pl.ds(ROWS_PT * NS, ROWS_REM)])
        plsc.subcore_barrier()

        pltpu.sync_copy(idx_hbm.at[c, s, 0], idx_r[0])
        pltpu.sync_copy(w_hbm.at[s, 0], w_r[0])
        stage_async(1, 1)
        issue_gathers(0, 0)
        section(0, 0, prefetch=True, prep=True, wait_prev=False)

        def quad(m, carry):
            for t in range(4):
                section(4 * m + 1 + t, 1 + t, prefetch=True, prep=True,
                        wait_prev=True)
            return carry
        lax.fori_loop(0, (NSUP - 3) // 4, quad, 0)

        section(NSUP - 2, NSUP - 2, prefetch=False, prep=True, wait_prev=True)
        section(NSUP - 1, NSUP - 1, prefetch=False, prep=False, wait_prev=False)
        wait_scatters((NSUP - 2) % NMSG, (NSUP - 2) % NIDX)
        wait_scatters((NSUP - 1) % NMSG, (NSUP - 1) % NIDX)

        plsc.subcore_barrier()

        pltpu.sync_copy(acc_sh.at[pl.ds(s * ROWS_PT, ROWS_PT)],
                        out_hbm.at[pl.ds(c * N + s * ROWS_PT, ROWS_PT)])

        @pl.when(s == 0)
        def _():
            pltpu.sync_copy(acc_sh.at[pl.ds(ROWS_PT * NS, ROWS_REM)],
                            out_hbm.at[pl.ds(c * N + ROWS_PT * NS, ROWS_REM)])

    return spmm


_spmm = _make_spmm()


def kernel(x, edge_index, edge_weight, W):
    dst = edge_index[0].astype(jnp.int32)
    src = edge_index[1].astype(jnp.int32)
    pad = E_PAD - E
    dst_p = jnp.concatenate([dst, jnp.zeros((pad,), jnp.int32)])
    src_p = jnp.concatenate([src, jnp.zeros((pad,), jnp.int32)])
    w_p = jnp.concatenate([edge_weight, jnp.zeros((pad,), jnp.float32)])

    dstr = dst_p.reshape(NS, NSUP, 2, CH)
    w_hbm = w_p.reshape(NS, NSUP, 2, CH)

    def pack(srcr):
        return jnp.stack([srcr[:, :, 0], dstr[:, :, 0],
                          srcr[:, :, 1], dstr[:, :, 1]], axis=2)

    s0 = src_p.reshape(NS, NSUP, 2, CH)
    idx_all = jnp.stack([pack(s0), pack(s0 + N)])

    h = _matmul_stacked(x, W)
    o = _spmm(h, idx_all, w_hbm)
    return jnp.concatenate([o[:N], o[N:]], axis=1)
